# baseline (speedup 1.0000x reference)
import functools
import jax
import jax.numpy as jnp
from jax.experimental import pallas as pl
from jax.experimental.pallas import tpu as pltpu

B = 8192
D_IN = 768
D_LATENT = 16384
K = 64


def _enc_body(x_ref, bpre_ref, encW_ref, encb_ref, a_ref):
    x0 = x_ref[...] - bpre_ref[...][None, :]
    w = encW_ref[...]
    acc = jax.lax.dot_general(
        x0, w, (((1,), (1,)), ((), ())), preferred_element_type=jnp.float32
    )
    a_ref[...] = acc + encb_ref[...][None, :]


def _dec_body(z_ref, decW_ref, decb_ref, o_ref):
    j = pl.program_id(1)
    acc = jax.lax.dot_general(
        z_ref[...], decW_ref[...], (((1,), (1,)), ((), ())),
        preferred_element_type=jnp.float32,
    )

    @pl.when(j == 0)
    def _():
        o_ref[...] = acc + decb_ref[...][None, :]

    @pl.when(j > 0)
    def _():
        o_ref[...] = o_ref[...] + acc


def _encode(x, b_pre, enc_W, enc_b):
    BB, LB = 512, 2048
    grid = (B // BB, D_LATENT // LB)
    return pl.pallas_call(
        _enc_body,
        grid=grid,
        in_specs=[
            pl.BlockSpec((BB, D_IN), lambda i, j: (i, 0)),
            pl.BlockSpec((D_IN,), lambda i, j: (0,)),
            pl.BlockSpec((LB, D_IN), lambda i, j: (j, 0)),
            pl.BlockSpec((LB,), lambda i, j: (j,)),
        ],
        out_specs=pl.BlockSpec((BB, LB), lambda i, j: (i, j)),
        out_shape=jax.ShapeDtypeStruct((B, D_LATENT), jnp.float32),
    )(x, b_pre, enc_W, enc_b)


def _decode(z, dec_W, dec_b):
    BB, LB = 1024, 2048
    grid = (B // BB, D_LATENT // LB)
    return pl.pallas_call(
        _dec_body,
        grid=grid,
        in_specs=[
            pl.BlockSpec((BB, LB), lambda i, j: (i, j)),
            pl.BlockSpec((D_IN, LB), lambda i, j: (0, j)),
            pl.BlockSpec((D_IN,), lambda i, j: (0,)),
        ],
        out_specs=pl.BlockSpec((BB, D_IN), lambda i, j: (i, 0)),
        out_shape=jax.ShapeDtypeStruct((B, D_IN), jnp.float32),
    )(z, dec_W, dec_b)


def kernel(x, b_pre, enc_W, enc_b, dec_W, dec_b):
    a = _encode(x, b_pre, enc_W, enc_b)
    a_relu = jax.nn.relu(a)
    topk_val, topk_idx = jax.lax.top_k(a_relu, K)
    rows = jnp.arange(B)[:, None]
    z = jnp.zeros_like(a_relu).at[rows, topk_idx].set(topk_val)
    x_hat = _decode(z, dec_W, dec_b)
    return (x_hat, z, a)


# speedup vs baseline: 15.4845x; 15.4845x over previous
import functools
import jax
import jax.numpy as jnp
from jax.experimental import pallas as pl
from jax.experimental.pallas import tpu as pltpu

B = 8192
D_IN = 768
D_LATENT = 16384
K = 64


def _enc_body(x_ref, bpre_ref, encW_ref, encb_ref, a_ref):
    x0 = x_ref[...] - bpre_ref[...][None, :]
    acc = jax.lax.dot_general(
        x0, encW_ref[...], (((1,), (1,)), ((), ())),
        preferred_element_type=jnp.float32,
    )
    a_ref[...] = acc + encb_ref[...][None, :]


def _encode(x, b_pre, enc_W, enc_b):
    BB, LB = 1024, 2048
    grid = (D_LATENT // LB, B // BB)
    return pl.pallas_call(
        _enc_body,
        grid=grid,
        in_specs=[
            pl.BlockSpec((BB, D_IN), lambda j, i: (i, 0)),
            pl.BlockSpec((D_IN,), lambda j, i: (0,)),
            pl.BlockSpec((LB, D_IN), lambda j, i: (j, 0)),
            pl.BlockSpec((LB,), lambda j, i: (j,)),
        ],
        out_specs=pl.BlockSpec((BB, LB), lambda j, i: (i, j)),
        out_shape=jax.ShapeDtypeStruct((B, D_LATENT), jnp.float32),
    )(x, b_pre, enc_W, enc_b)


def _select_body(a_ref, z_ref):
    a = a_ref[...]
    bits = jax.lax.bitcast_convert_type(jnp.maximum(a, 0.0), jnp.int32)
    t = jnp.zeros((a.shape[0], 1), jnp.int32)
    for b in range(30, -1, -1):
        cand = t | (1 << b)
        cnt = jnp.sum((bits >= cand).astype(jnp.int32), axis=1, keepdims=True)
        t = jnp.where(cnt >= K, cand, t)
    t = jnp.maximum(t, 1)
    z_ref[...] = jnp.where(bits >= t, a, 0.0)


def _select(a):
    BB = 128
    return pl.pallas_call(
        _select_body,
        grid=(B // BB,),
        in_specs=[pl.BlockSpec((BB, D_LATENT), lambda i: (i, 0))],
        out_specs=pl.BlockSpec((BB, D_LATENT), lambda i: (i, 0)),
        out_shape=jax.ShapeDtypeStruct((B, D_LATENT), jnp.float32),
    )(a)


def _dec_body(z_ref, decW_ref, decb_ref, o_ref):
    j = pl.program_id(1)
    acc = jax.lax.dot_general(
        z_ref[...], decW_ref[...], (((1,), (1,)), ((), ())),
        preferred_element_type=jnp.float32,
    )

    @pl.when(j == 0)
    def _():
        o_ref[...] = acc + decb_ref[...][None, :]

    @pl.when(j > 0)
    def _():
        o_ref[...] = o_ref[...] + acc


def _decode(z, dec_W, dec_b):
    BB, LB = 2048, 1024
    grid = (B // BB, D_LATENT // LB)
    return pl.pallas_call(
        _dec_body,
        grid=grid,
        in_specs=[
            pl.BlockSpec((BB, LB), lambda i, j: (i, j)),
            pl.BlockSpec((D_IN, LB), lambda i, j: (0, j)),
            pl.BlockSpec((D_IN,), lambda i, j: (0,)),
        ],
        out_specs=pl.BlockSpec((BB, D_IN), lambda i, j: (i, 0)),
        out_shape=jax.ShapeDtypeStruct((B, D_IN), jnp.float32),
    )(z, dec_W, dec_b)


def kernel(x, b_pre, enc_W, enc_b, dec_W, dec_b):
    a = _encode(x, b_pre, enc_W, enc_b)
    z = _select(a)
    x_hat = _decode(z, dec_W, dec_b)
    return (x_hat, z, a)
